# baseline (speedup 1.0000x reference)
import jax
import jax.numpy as jnp
from jax.experimental import pallas as pl
from jax.experimental.pallas import tpu as pltpu

_N_FIELDS = 26
_EMB = 64
_DM = 512
_STEPS = _N_FIELDS + 1


def _row_copy(idxs_ref, uid_hbm, cat_hbm, emb_ref, sems, step, slot):
    idx = idxs_ref[step]

    def _start_uid():
        pltpu.make_async_copy(
            uid_hbm.at[pl.ds(idx, 1), :], emb_ref.at[pl.ds(slot, 1), :],
            sems.at[slot]).start()

    def _start_cat():
        pltpu.make_async_copy(
            cat_hbm.at[step - 1, pl.ds(idx, 1), :],
            emb_ref.at[pl.ds(slot, 1), :], sems.at[slot]).start()

    jax.lax.cond(step == 0, _start_uid, _start_cat)


def _mlp_body(idxs_ref, uid_hbm, cat_hbm, w1_ref, b1_ref, w2_ref, b2_ref,
              out_ref, emb_ref, acc_ref, sems):
    i = pl.program_id(0)

    @pl.when(i == 0)
    def _prologue():
        _row_copy(idxs_ref, uid_hbm, cat_hbm, emb_ref, sems, 0, 0)

    @pl.when(i < _STEPS - 1)
    def _prefetch():
        _row_copy(idxs_ref, uid_hbm, cat_hbm, emb_ref, sems, i + 1,
                  (i + 1) % 2)

    pltpu.make_async_copy(
        uid_hbm.at[pl.ds(0, 1), :], emb_ref.at[pl.ds(i % 2, 1), :],
        sems.at[i % 2]).wait()

    emb = emb_ref[pl.ds(i % 2, 1), :]
    partial = jnp.dot(emb, w1_ref[...], preferred_element_type=jnp.float32)

    @pl.when(i == 0)
    def _init():
        acc_ref[...] = partial

    @pl.when(i > 0)
    def _accum():
        acc_ref[...] += partial

    @pl.when(i == _STEPS - 1)
    def _finish():
        x = acc_ref[...] + b1_ref[...]
        x = jnp.where(x >= 0, x, 0.01 * x)
        out_ref[...] = (jnp.dot(x, w2_ref[...], preferred_element_type=jnp.float32)
                        + b2_ref[...])


def kernel(uid, onehot_feats, uid_table, cat_tables, W1, b1, W2, b2):
    idxs = jnp.concatenate(
        [uid.astype(jnp.int32), onehot_feats.reshape(-1).astype(jnp.int32)])

    out = pl.pallas_call(
        _mlp_body,
        grid=(_STEPS,),
        in_specs=[
            pl.BlockSpec(memory_space=pltpu.SMEM),
            pl.BlockSpec(memory_space=pl.ANY),
            pl.BlockSpec(memory_space=pl.ANY),
            pl.BlockSpec((_EMB, _DM), lambda i: (i, 0)),
            pl.BlockSpec((1, _DM), lambda i: (0, 0)),
            pl.BlockSpec((_DM, _DM), lambda i: (0, 0)),
            pl.BlockSpec((1, _DM), lambda i: (0, 0)),
        ],
        out_specs=pl.BlockSpec((1, _DM), lambda i: (0, 0)),
        out_shape=jax.ShapeDtypeStruct((1, _DM), jnp.float32),
        scratch_shapes=[
            pltpu.VMEM((2, _EMB), jnp.float32),
            pltpu.VMEM((1, _DM), jnp.float32),
            pltpu.SemaphoreType.DMA((2,)),
        ],
    )(idxs, uid_table, cat_tables, W1, b1.reshape(1, -1), W2,
      b2.reshape(1, -1))
    return out[None]


# speedup vs baseline: 54.3017x; 54.3017x over previous
import jax
import jax.numpy as jnp
from jax.experimental import pallas as pl
from jax.experimental.pallas import tpu as pltpu

_N_FIELDS = 26
_EMB = 64
_DM = 512
_LANES = 128
_STEPS = _N_FIELDS + 1


def _tile_copy(idxs_ref, uid_hbm, cat_hbm, emb_ref, sems, step, slot):
    base = (idxs_ref[step] // _LANES) * _LANES

    def _start_uid():
        pltpu.make_async_copy(
            uid_hbm.at[:, pl.ds(base, _LANES)], emb_ref.at[slot],
            sems.at[slot]).start()

    def _start_cat():
        pltpu.make_async_copy(
            cat_hbm.at[step - 1, :, pl.ds(base, _LANES)], emb_ref.at[slot],
            sems.at[slot]).start()

    jax.lax.cond(step == 0, _start_uid, _start_cat)


def _mlp_body(idxs_ref, uid_hbm, cat_hbm, w1_ref, b1_ref, w2_ref, b2_ref,
              out_ref, emb_ref, acc_ref, sems):
    i = pl.program_id(0)

    @pl.when(i == 0)
    def _prologue():
        _tile_copy(idxs_ref, uid_hbm, cat_hbm, emb_ref, sems, 0, 0)

    @pl.when(i < _STEPS - 1)
    def _prefetch():
        _tile_copy(idxs_ref, uid_hbm, cat_hbm, emb_ref, sems, i + 1,
                   (i + 1) % 2)

    pltpu.make_async_copy(
        uid_hbm.at[:, pl.ds(0, _LANES)], emb_ref.at[i % 2],
        sems.at[i % 2]).wait()

    lane = idxs_ref[i] % _LANES
    tile = emb_ref[i % 2]
    mask = jax.lax.broadcasted_iota(jnp.int32, (_EMB, _LANES), 1) == lane
    col = jnp.sum(jnp.where(mask, tile, 0.0), axis=1, keepdims=True)

    partial = jax.lax.dot_general(
        col, w1_ref[...], (((0,), (0,)), ((), ())),
        preferred_element_type=jnp.float32)

    @pl.when(i == 0)
    def _init():
        acc_ref[...] = partial

    @pl.when(i > 0)
    def _accum():
        acc_ref[...] += partial

    @pl.when(i == _STEPS - 1)
    def _finish():
        x = acc_ref[...] + b1_ref[...]
        x = jnp.where(x >= 0, x, 0.01 * x)
        out_ref[...] = (jnp.dot(x, w2_ref[...], preferred_element_type=jnp.float32)
                        + b2_ref[...])


def kernel(uid, onehot_feats, uid_table, cat_tables, W1, b1, W2, b2):
    idxs = jnp.concatenate(
        [uid.astype(jnp.int32), onehot_feats.reshape(-1).astype(jnp.int32)])
    uid_t = uid_table.T
    cat_t = jnp.transpose(cat_tables, (0, 2, 1))

    out = pl.pallas_call(
        _mlp_body,
        grid=(_STEPS,),
        in_specs=[
            pl.BlockSpec(memory_space=pltpu.SMEM),
            pl.BlockSpec(memory_space=pl.ANY),
            pl.BlockSpec(memory_space=pl.ANY),
            pl.BlockSpec((_EMB, _DM), lambda i: (i, 0)),
            pl.BlockSpec((1, _DM), lambda i: (0, 0)),
            pl.BlockSpec((_DM, _DM), lambda i: (0, 0)),
            pl.BlockSpec((1, _DM), lambda i: (0, 0)),
        ],
        out_specs=pl.BlockSpec((1, _DM), lambda i: (0, 0)),
        out_shape=jax.ShapeDtypeStruct((1, _DM), jnp.float32),
        scratch_shapes=[
            pltpu.VMEM((2, _EMB, _LANES), jnp.float32),
            pltpu.VMEM((1, _DM), jnp.float32),
            pltpu.SemaphoreType.DMA((2,)),
        ],
    )(idxs, uid_t, cat_t, W1, b1.reshape(1, -1), W2, b2.reshape(1, -1))
    return out[None]


# speedup vs baseline: 59.3108x; 1.0922x over previous
import jax
import jax.numpy as jnp
from jax.experimental import pallas as pl
from jax.experimental.pallas import tpu as pltpu

_N_FIELDS = 26
_EMB = 64
_DM = 512
_LANES = 128
_STEPS = _N_FIELDS + 1


def _mlp_body(idxs_ref, uid_hbm, cat_hbm, w1_ref, b1_ref, w2_ref, b2_ref,
              out_ref, emb_ref, acc_ref, sems):
    i = pl.program_id(0)

    @pl.when(i == 0)
    def _prologue():
        base0 = (idxs_ref[0] // _LANES) * _LANES
        pltpu.make_async_copy(
            uid_hbm.at[:, pl.ds(base0, _LANES)], emb_ref.at[0],
            sems.at[0]).start()
        for s in range(1, _STEPS):
            base = (idxs_ref[s] // _LANES) * _LANES
            pltpu.make_async_copy(
                cat_hbm.at[s - 1, :, pl.ds(base, _LANES)], emb_ref.at[s],
                sems.at[s]).start()

    pltpu.make_async_copy(
        uid_hbm.at[:, pl.ds(0, _LANES)], emb_ref.at[i],
        sems.at[i]).wait()

    lane = idxs_ref[i] % _LANES
    tile = emb_ref[i]
    mask = jax.lax.broadcasted_iota(jnp.int32, (_EMB, _LANES), 1) == lane
    col = jnp.sum(jnp.where(mask, tile, 0.0), axis=1, keepdims=True)

    partial = jax.lax.dot_general(
        col, w1_ref[...], (((0,), (0,)), ((), ())),
        preferred_element_type=jnp.float32)

    @pl.when(i == 0)
    def _init():
        acc_ref[...] = partial

    @pl.when(i > 0)
    def _accum():
        acc_ref[...] += partial

    @pl.when(i == _STEPS - 1)
    def _finish():
        x = acc_ref[...] + b1_ref[...]
        x = jnp.where(x >= 0, x, 0.01 * x)
        out_ref[...] = (jnp.dot(x, w2_ref[...], preferred_element_type=jnp.float32)
                        + b2_ref[...])


def kernel(uid, onehot_feats, uid_table, cat_tables, W1, b1, W2, b2):
    idxs = jnp.concatenate(
        [uid.astype(jnp.int32), onehot_feats.reshape(-1).astype(jnp.int32)])
    uid_t = uid_table.T
    cat_t = jnp.transpose(cat_tables, (0, 2, 1))

    out = pl.pallas_call(
        _mlp_body,
        grid=(_STEPS,),
        in_specs=[
            pl.BlockSpec(memory_space=pltpu.SMEM),
            pl.BlockSpec(memory_space=pl.ANY),
            pl.BlockSpec(memory_space=pl.ANY),
            pl.BlockSpec((_EMB, _DM), lambda i: (i, 0)),
            pl.BlockSpec((1, _DM), lambda i: (0, 0)),
            pl.BlockSpec((_DM, _DM), lambda i: (0, 0)),
            pl.BlockSpec((1, _DM), lambda i: (0, 0)),
        ],
        out_specs=pl.BlockSpec((1, _DM), lambda i: (0, 0)),
        out_shape=jax.ShapeDtypeStruct((1, _DM), jnp.float32),
        scratch_shapes=[
            pltpu.VMEM((_STEPS, _EMB, _LANES), jnp.float32),
            pltpu.VMEM((1, _DM), jnp.float32),
            pltpu.SemaphoreType.DMA((_STEPS,)),
        ],
    )(idxs, uid_t, cat_t, W1, b1.reshape(1, -1), W2, b2.reshape(1, -1))
    return out[None]


# speedup vs baseline: 119.8726x; 2.0211x over previous
import jax
import jax.numpy as jnp
from jax.experimental import pallas as pl
from jax.experimental.pallas import tpu as pltpu

_N_FIELDS = 26
_EMB = 64
_DM = 512
_LANES = 128
_STEPS = _N_FIELDS + 1


def _mlp_body(idxs_ref, uid_hbm, cat_hbm, w1_ref, b1_ref, w2_ref, b2_ref,
              out_ref, emb_ref, sems):
    base0 = (idxs_ref[0] // _LANES) * _LANES
    pltpu.make_async_copy(
        uid_hbm.at[:, pl.ds(base0, _LANES)], emb_ref.at[0], sems.at[0]).start()
    for s in range(1, _STEPS):
        base = (idxs_ref[s] // _LANES) * _LANES
        pltpu.make_async_copy(
            cat_hbm.at[s - 1, :, pl.ds(base, _LANES)], emb_ref.at[s],
            sems.at[s]).start()

    lane_iota = jax.lax.broadcasted_iota(jnp.int32, (_EMB, _LANES), 1)
    acc = None
    for s in range(_STEPS):
        pltpu.make_async_copy(
            uid_hbm.at[:, pl.ds(0, _LANES)], emb_ref.at[s], sems.at[s]).wait()
        lane = idxs_ref[s] % _LANES
        tile = emb_ref[s]
        col = jnp.sum(jnp.where(lane_iota == lane, tile, 0.0), axis=1,
                      keepdims=True)
        partial = jax.lax.dot_general(
            col, w1_ref[pl.ds(s * _EMB, _EMB), :], (((0,), (0,)), ((), ())),
            preferred_element_type=jnp.float32)
        acc = partial if acc is None else acc + partial

    x = acc + b1_ref[...]
    x = jnp.where(x >= 0, x, 0.01 * x)
    out_ref[...] = (jnp.dot(x, w2_ref[...], preferred_element_type=jnp.float32)
                    + b2_ref[...])


def kernel(uid, onehot_feats, uid_table, cat_tables, W1, b1, W2, b2):
    idxs = jnp.concatenate(
        [uid.astype(jnp.int32), onehot_feats.reshape(-1).astype(jnp.int32)])
    uid_t = uid_table.T
    cat_t = jnp.transpose(cat_tables, (0, 2, 1))

    out = pl.pallas_call(
        _mlp_body,
        in_specs=[
            pl.BlockSpec(memory_space=pltpu.SMEM),
            pl.BlockSpec(memory_space=pl.ANY),
            pl.BlockSpec(memory_space=pl.ANY),
            pl.BlockSpec(memory_space=pltpu.VMEM),
            pl.BlockSpec(memory_space=pltpu.VMEM),
            pl.BlockSpec(memory_space=pltpu.VMEM),
            pl.BlockSpec(memory_space=pltpu.VMEM),
        ],
        out_specs=pl.BlockSpec(memory_space=pltpu.VMEM),
        out_shape=jax.ShapeDtypeStruct((1, _DM), jnp.float32),
        scratch_shapes=[
            pltpu.VMEM((_STEPS, _EMB, _LANES), jnp.float32),
            pltpu.SemaphoreType.DMA((_STEPS,)),
        ],
    )(idxs, uid_t, cat_t, W1, b1.reshape(1, -1), W2, b2.reshape(1, -1))
    return out[None]


# speedup vs baseline: 141.6549x; 1.1817x over previous
import jax
import jax.numpy as jnp
from jax.experimental import pallas as pl
from jax.experimental.pallas import tpu as pltpu

_N_FIELDS = 26
_EMB = 64
_DM = 512
_LANES = 128
_STEPS = _N_FIELDS + 1


def _mlp_body(uid_ref, feats_ref, uid_hbm, cat_hbm, w1_ref, b1_ref, w2_ref,
              b2_ref, out_ref, emb_ref, fu_ref, sems):
    def _idx(s):
        return uid_ref[0] if s == 0 else feats_ref[s - 1, 0]

    base0 = (_idx(0) // _LANES) * _LANES
    pltpu.make_async_copy(
        uid_hbm.at[:, pl.ds(base0, _LANES)], emb_ref.at[0], sems.at[0]).start()
    for s in range(1, _STEPS):
        base = (_idx(s) // _LANES) * _LANES
        pltpu.make_async_copy(
            cat_hbm.at[s - 1, :, pl.ds(base, _LANES)], emb_ref.at[s],
            sems.at[s]).start()

    lane_iota = jax.lax.broadcasted_iota(jnp.int32, (_EMB, _LANES), 1)
    for s in range(_STEPS):
        pltpu.make_async_copy(
            uid_hbm.at[:, pl.ds(0, _LANES)], emb_ref.at[s], sems.at[s]).wait()
        lane = _idx(s) % _LANES
        tile = emb_ref[s]
        col = jnp.sum(jnp.where(lane_iota == lane, tile, 0.0), axis=1,
                      keepdims=True)
        fu_ref[pl.ds(s * _EMB, _EMB), :] = col

    x = jax.lax.dot_general(
        fu_ref[...], w1_ref[...], (((0,), (0,)), ((), ())),
        preferred_element_type=jnp.float32) + b1_ref[...]
    x = jnp.where(x >= 0, x, 0.01 * x)
    out_ref[...] = (jnp.dot(x, w2_ref[...], preferred_element_type=jnp.float32)
                    + b2_ref[...])


def kernel(uid, onehot_feats, uid_table, cat_tables, W1, b1, W2, b2):
    uid_t = uid_table.T
    cat_t = jnp.transpose(cat_tables, (0, 2, 1))

    out = pl.pallas_call(
        _mlp_body,
        in_specs=[
            pl.BlockSpec(memory_space=pltpu.SMEM),
            pl.BlockSpec(memory_space=pltpu.SMEM),
            pl.BlockSpec(memory_space=pl.ANY),
            pl.BlockSpec(memory_space=pl.ANY),
            pl.BlockSpec(memory_space=pltpu.VMEM),
            pl.BlockSpec(memory_space=pltpu.VMEM),
            pl.BlockSpec(memory_space=pltpu.VMEM),
            pl.BlockSpec(memory_space=pltpu.VMEM),
        ],
        out_specs=pl.BlockSpec(memory_space=pltpu.VMEM),
        out_shape=jax.ShapeDtypeStruct((1, _DM), jnp.float32),
        scratch_shapes=[
            pltpu.VMEM((_STEPS, _EMB, _LANES), jnp.float32),
            pltpu.VMEM((_STEPS * _EMB, 1), jnp.float32),
            pltpu.SemaphoreType.DMA((_STEPS,)),
        ],
    )(uid.astype(jnp.int32), onehot_feats.astype(jnp.int32), uid_t, cat_t,
      W1, b1.reshape(1, -1), W2, b2.reshape(1, -1))
    return out[None]


# speedup vs baseline: 161.2971x; 1.1387x over previous
import jax
import jax.numpy as jnp
from jax.experimental import pallas as pl
from jax.experimental.pallas import tpu as pltpu

_N_FIELDS = 26
_EMB = 64
_DM = 512
_LANES = 128
_STEPS = _N_FIELDS + 1


def _mlp_body(uid_ref, feats_ref, uid_hbm, cat_hbm, w1_hbm, b1_hbm, w2_hbm,
              b2_hbm, out_ref, emb_ref, fu_ref, w1_ref, b1_ref, w2_ref,
              b2_ref, sems, wsems):
    def _idx(s):
        return uid_ref[0] if s == 0 else feats_ref[s - 1, 0]

    w1_dma = pltpu.make_async_copy(w1_hbm, w1_ref, wsems.at[0])
    w2_dma = pltpu.make_async_copy(w2_hbm, w2_ref, wsems.at[1])
    b1_dma = pltpu.make_async_copy(b1_hbm, b1_ref, wsems.at[2])
    b2_dma = pltpu.make_async_copy(b2_hbm, b2_ref, wsems.at[3])
    w1_dma.start()
    w2_dma.start()
    b1_dma.start()
    b2_dma.start()

    base0 = (_idx(0) // _LANES) * _LANES
    pltpu.make_async_copy(
        uid_hbm.at[:, pl.ds(base0, _LANES)], emb_ref.at[0], sems.at[0]).start()
    for s in range(1, _STEPS):
        base = (_idx(s) // _LANES) * _LANES
        pltpu.make_async_copy(
            cat_hbm.at[s - 1, :, pl.ds(base, _LANES)], emb_ref.at[s],
            sems.at[s]).start()

    lane_iota = jax.lax.broadcasted_iota(jnp.int32, (_EMB, _LANES), 1)
    for s in range(_STEPS):
        pltpu.make_async_copy(
            uid_hbm.at[:, pl.ds(0, _LANES)], emb_ref.at[s], sems.at[s]).wait()
        lane = _idx(s) % _LANES
        tile = emb_ref[s]
        col = jnp.sum(jnp.where(lane_iota == lane, tile, 0.0), axis=1,
                      keepdims=True)
        fu_ref[pl.ds(s * _EMB, _EMB), :] = col

    w1_dma.wait()
    b1_dma.wait()
    x = jax.lax.dot_general(
        fu_ref[...], w1_ref[...], (((0,), (0,)), ((), ())),
        preferred_element_type=jnp.float32) + b1_ref[...]
    x = jnp.where(x >= 0, x, 0.01 * x)
    w2_dma.wait()
    b2_dma.wait()
    out_ref[...] = (jnp.dot(x, w2_ref[...], preferred_element_type=jnp.float32)
                    + b2_ref[...])


def kernel(uid, onehot_feats, uid_table, cat_tables, W1, b1, W2, b2):
    uid_t = uid_table.T
    cat_t = jnp.transpose(cat_tables, (0, 2, 1))

    out = pl.pallas_call(
        _mlp_body,
        in_specs=[
            pl.BlockSpec(memory_space=pltpu.SMEM),
            pl.BlockSpec(memory_space=pltpu.SMEM),
            pl.BlockSpec(memory_space=pl.ANY),
            pl.BlockSpec(memory_space=pl.ANY),
            pl.BlockSpec(memory_space=pl.ANY),
            pl.BlockSpec(memory_space=pl.ANY),
            pl.BlockSpec(memory_space=pl.ANY),
            pl.BlockSpec(memory_space=pl.ANY),
        ],
        out_specs=pl.BlockSpec(memory_space=pltpu.VMEM),
        out_shape=jax.ShapeDtypeStruct((1, _DM), jnp.float32),
        scratch_shapes=[
            pltpu.VMEM((_STEPS, _EMB, _LANES), jnp.float32),
            pltpu.VMEM((_STEPS * _EMB, 1), jnp.float32),
            pltpu.VMEM((_STEPS * _EMB, _DM), jnp.float32),
            pltpu.VMEM((1, _DM), jnp.float32),
            pltpu.VMEM((_DM, _DM), jnp.float32),
            pltpu.VMEM((1, _DM), jnp.float32),
            pltpu.SemaphoreType.DMA((_STEPS,)),
            pltpu.SemaphoreType.DMA((4,)),
        ],
    )(uid.astype(jnp.int32), onehot_feats.astype(jnp.int32), uid_t, cat_t,
      W1, b1.reshape(1, -1), W2, b2.reshape(1, -1))
    return out[None]
